# baseline (speedup 1.0000x reference)
import jax
import jax.numpy as jnp
from jax.experimental import pallas as pl
from jax.experimental.pallas import tpu as pltpu

_NLAYER = 3
_L = 24
_LPAD = 32
_NTILE = 8


def _gnn_body(xn_ref, adj_hbm, ws_hbm, wn_hbm, b_hbm, u_ref, v_ref,
              adj_ref, ws_ref, wn_ref, b_ref, ab_ref, dinv_ref, sems):
    i = pl.program_id(0)

    @pl.when(i == 0)
    def _():
        pltpu.make_async_copy(adj_hbm, adj_ref, sems.at[0]).start()
        pltpu.make_async_copy(ws_hbm, ws_ref, sems.at[1]).start()
        pltpu.make_async_copy(wn_hbm, wn_ref, sems.at[2]).start()
        pltpu.make_async_copy(b_hbm, b_ref, sems.at[3]).start()
        pltpu.make_async_copy(adj_hbm, adj_ref, sems.at[0]).wait()
        pltpu.make_async_copy(ws_hbm, ws_ref, sems.at[1]).wait()
        pltpu.make_async_copy(wn_hbm, wn_ref, sems.at[2]).wait()
        pltpu.make_async_copy(b_hbm, b_ref, sems.at[3]).wait()
        A0 = adj_ref[...]
        Ab = (A0 != 0).astype(jnp.float32)
        deg = jnp.sum(Ab, axis=0)
        deg_inv = jnp.where(deg > 0, 1.0 / jnp.maximum(deg, 1.0), 0.0)
        ab_ref[...] = Ab.astype(jnp.bfloat16)
        dinv_ref[...] = deg_inv[:, None]

    Ab = ab_ref[...]
    dinv = dinv_ref[...]

    def dotT_bf(Lhs, H):
        hi = H.astype(jnp.bfloat16)
        lo = (H - hi.astype(jnp.float32)).astype(jnp.bfloat16)
        dn = (((0,), (0,)), ((), ()))
        return (jax.lax.dot_general(Lhs, hi, dn,
                                    preferred_element_type=jnp.float32)
                + jax.lax.dot_general(Lhs, lo, dn,
                                      preferred_element_type=jnp.float32))

    Xc = xn_ref[...]
    U = 4.0 * Xc
    V = dotT_bf(Ab, Xc)
    for l in range(_NLAYER):
        Ws = ws_ref[l]
        Wn = wn_ref[l]
        bias = b_ref[l]
        AU = dotT_bf(Ab, U) * dinv
        AV = dotT_bf(Ab, V) * dinv
        U = jnp.dot(U, Ws, preferred_element_type=jnp.float32) \
            + jnp.dot(AU, Wn, preferred_element_type=jnp.float32) + bias
        V = jnp.dot(V, Ws, preferred_element_type=jnp.float32) \
            + jnp.dot(AV, Wn, preferred_element_type=jnp.float32) + bias
    u_ref[...] = U
    v_ref[...] = V


def kernel(x, adj, W_self, b_self, W_neigh):
    nS, nC, nN, L = x.shape
    nG = nC * nS

    def mk_tiles(W):
        Wp = jnp.pad(jnp.swapaxes(W, 1, 2),
                     ((0, 0), (0, _LPAD - L), (0, _LPAD - L)))
        z = jnp.zeros_like(Wp)
        rows = [jnp.concatenate([Wp if c == r else z for c in range(4)], axis=2)
                for r in range(4)]
        return jnp.concatenate(rows, axis=1)

    Wst = mk_tiles(W_self)
    Wnt = mk_tiles(W_neigh)
    bp = jnp.pad(b_self, ((0, 0), (0, _LPAD - L)))
    bt = jnp.tile(bp, (1, 4)).reshape(_NLAYER, 1, 128)

    Xn = jnp.transpose(x, (2, 0, 1, 3))
    Xn = jnp.pad(Xn, ((0, 0), (0, 0), (0, 0), (0, _LPAD - L)))
    Xn = Xn.reshape(nN, nG * _LPAD)

    U, V = pl.pallas_call(
        _gnn_body,
        grid=(_NTILE,),
        in_specs=[
            pl.BlockSpec((nN, 128), lambda i: (0, i)),
            pl.BlockSpec(memory_space=pl.ANY),
            pl.BlockSpec(memory_space=pl.ANY),
            pl.BlockSpec(memory_space=pl.ANY),
            pl.BlockSpec(memory_space=pl.ANY),
        ],
        out_specs=[
            pl.BlockSpec((nN, 128), lambda i: (0, i)),
            pl.BlockSpec((nN, 128), lambda i: (0, i)),
        ],
        out_shape=[
            jax.ShapeDtypeStruct((nN, nG * _LPAD), jnp.float32),
            jax.ShapeDtypeStruct((nN, nG * _LPAD), jnp.float32),
        ],
        scratch_shapes=[
            pltpu.VMEM((nN, nN), jnp.float32),
            pltpu.VMEM((_NLAYER, 128, 128), jnp.float32),
            pltpu.VMEM((_NLAYER, 128, 128), jnp.float32),
            pltpu.VMEM((_NLAYER, 1, 128), jnp.float32),
            pltpu.VMEM((nN, nN), jnp.bfloat16),
            pltpu.VMEM((nN, 1), jnp.float32),
            pltpu.SemaphoreType.DMA((4,)),
        ],
    )(Xn, adj, Wst, Wnt, bt)

    Ur = U.reshape(nN, nS, nC, _LPAD)[..., :L].transpose(1, 2, 0, 3)
    Vr = V.reshape(nN, nS, nC, _LPAD)[..., :L].transpose(1, 2, 0, 3)
    out = jnp.stack([Ur, Vr], axis=2).reshape(nS, 2 * nC, nN, L)
    return out


# speedup vs baseline: 1.4708x; 1.4708x over previous
import jax
import jax.numpy as jnp
from jax.experimental import pallas as pl

_NLAYER = 3
_L = 24
_LPAD = 32
_NTILE = 8


def _gnn_body(xn_ref, adj_ref, ws_ref, wn_ref, b_ref, u_ref, v_ref):
    A = adj_ref[...]
    Ab = (A != 0).astype(jnp.bfloat16)
    deg = jnp.sum(A, axis=0)
    deg_inv = jnp.where(deg > 0, 1.0 / jnp.maximum(deg, 1.0), 0.0)

    def dotT_bf(Lhs, H):
        hi = H.astype(jnp.bfloat16)
        lo = (H - hi.astype(jnp.float32)).astype(jnp.bfloat16)
        dn = (((0,), (0,)), ((), ()))
        return (jax.lax.dot_general(Lhs, hi, dn,
                                    preferred_element_type=jnp.float32)
                + jax.lax.dot_general(Lhs, lo, dn,
                                      preferred_element_type=jnp.float32))

    def lin(H, W):
        cols = [
            jnp.dot(H[:, 128 * t:128 * (t + 1)], W,
                    preferred_element_type=jnp.float32)
            for t in range(_NTILE)
        ]
        return jnp.concatenate(cols, axis=1)

    Xn = xn_ref[...]
    U = 4.0 * Xn
    V = dotT_bf(Ab, Xn)
    dcol = deg_inv[:, None]
    for i in range(_NLAYER):
        Ws = ws_ref[i]
        Wn = wn_ref[i]
        bias = b_ref[i]
        AU = dotT_bf(Ab, U) * dcol
        AV = dotT_bf(Ab, V) * dcol
        U = lin(U, Ws) + lin(AU, Wn) + bias[None, :]
        V = lin(V, Ws) + lin(AV, Wn) + bias[None, :]
    u_ref[...] = U
    v_ref[...] = V


def kernel(x, adj, W_self, b_self, W_neigh):
    nS, nC, nN, L = x.shape
    nG = nC * nS

    def mk_tiles(W):
        Wp = jnp.pad(jnp.swapaxes(W, 1, 2),
                     ((0, 0), (0, _LPAD - L), (0, _LPAD - L)))
        z = jnp.zeros_like(Wp)
        rows = [jnp.concatenate([Wp if c == r else z for c in range(4)], axis=2)
                for r in range(4)]
        return jnp.concatenate(rows, axis=1)

    Wst = mk_tiles(W_self)
    Wnt = mk_tiles(W_neigh)
    bt = jnp.tile(jnp.pad(b_self, ((0, 0), (0, _LPAD - L))), (1, nG))

    Xn = jnp.transpose(x, (2, 0, 1, 3))
    Xn = jnp.pad(Xn, ((0, 0), (0, 0), (0, 0), (0, _LPAD - L)))
    Xn = Xn.reshape(nN, nG * _LPAD)

    U, V = pl.pallas_call(
        _gnn_body,
        out_shape=[
            jax.ShapeDtypeStruct((nN, nG * _LPAD), jnp.float32),
            jax.ShapeDtypeStruct((nN, nG * _LPAD), jnp.float32),
        ],
    )(Xn, adj, Wst, Wnt, bt)

    Ur = U.reshape(nN, nS, nC, _LPAD)[..., :L].transpose(1, 2, 0, 3)
    Vr = V.reshape(nN, nS, nC, _LPAD)[..., :L].transpose(1, 2, 0, 3)
    out = jnp.stack([Ur, Vr], axis=2).reshape(nS, 2 * nC, nN, L)
    return out
